# Initial kernel scaffold; baseline (speedup 1.0000x reference)
import jax
import jax.numpy as jnp
from jax.experimental import pallas as pl


def kernel(x, x_mask, h, edge_index, W_in, b_in, U):
    raise NotImplementedError("write your pallas kernel here")


# speedup vs baseline: 3.6951x; 3.6951x over previous
import functools

import jax
import jax.numpy as jnp
from jax import lax
from jax.experimental import pallas as pl
from jax.experimental.pallas import tpu as pltpu
from jax.experimental.pallas import tpu_sc as plsc

N_NODES = 10000
N_EDGES = 320000
HDIM = 128

NC = 2
NS = 16
CHUNK = 128
NCHUNKS = 80
EDGES_PAD = NC * NS * NCHUNKS * CHUNK
ACC_ROWS = 10112
ZROWS = ACC_ROWS // NS


def _sc_segment_sum(h, src, dst, zeros):
    mesh = plsc.VectorSubcoreMesh(core_axis_name="c", subcore_axis_name="s")

    @functools.partial(
        pl.kernel,
        out_type=jax.ShapeDtypeStruct((NC, ACC_ROWS, HDIM), jnp.float32),
        mesh=mesh,
        scratch_types=[
            pltpu.VMEM((NCHUNKS, CHUNK), jnp.int32),
            pltpu.VMEM((NCHUNKS, CHUNK), jnp.int32),
            pltpu.VMEM((CHUNK, HDIM), jnp.float32),
            pltpu.VMEM_SHARED((ACC_ROWS, HDIM), jnp.float32),
            pltpu.SemaphoreType.DMA,
        ],
    )
    def k(h_hbm, src_hbm, dst_hbm, zero_hbm, out_hbm, src_v, dst_v, rows_v, acc, sem):
        cid = lax.axis_index("c")
        sid = lax.axis_index("s")

        pltpu.sync_copy(zero_hbm.at[pl.ds(sid * ZROWS, ZROWS)],
                        acc.at[pl.ds(sid * ZROWS, ZROWS)])
        pltpu.sync_copy(src_hbm.at[cid, sid], src_v)
        pltpu.sync_copy(dst_hbm.at[cid, sid], dst_v)
        plsc.subcore_barrier()

        def body(j, carry):
            pltpu.async_copy(h_hbm.at[src_v.at[j]], rows_v, sem).wait()
            pltpu.sync_copy(rows_v, acc.at[dst_v.at[j]], add=True)
            return carry

        lax.fori_loop(0, NCHUNKS, body, 0, unroll=False)

        plsc.subcore_barrier()
        pltpu.sync_copy(acc.at[pl.ds(sid * ZROWS, ZROWS)],
                        out_hbm.at[cid, pl.ds(sid * ZROWS, ZROWS)])

    return k(h, src, dst, zeros)


def _dense_body(x_ref, m_ref, p0_ref, p1_ref, w_ref, b_ref, u_ref, o_ref):
    hsum = p0_ref[...] + p1_ref[...]
    h_aggr = jnp.dot(hsum, u_ref[...], preferred_element_type=jnp.float32)
    h_in = jnp.dot(x_ref[...], w_ref[...], preferred_element_type=jnp.float32)
    h_in = (h_in + b_ref[...]) * m_ref[...]
    o_ref[...] = jnp.tanh(h_in + h_aggr)


def _dense_stage(x, mask2d, p0, p1, W_in, b2d, U):
    R = 1000
    grid = (N_NODES // R,)
    return pl.pallas_call(
        _dense_body,
        grid=grid,
        in_specs=[
            pl.BlockSpec((R, HDIM), lambda i: (i, 0)),
            pl.BlockSpec((R, 1), lambda i: (i, 0)),
            pl.BlockSpec((R, HDIM), lambda i: (i, 0)),
            pl.BlockSpec((R, HDIM), lambda i: (i, 0)),
            pl.BlockSpec((HDIM, HDIM), lambda i: (0, 0)),
            pl.BlockSpec((1, HDIM), lambda i: (0, 0)),
            pl.BlockSpec((HDIM, HDIM), lambda i: (0, 0)),
        ],
        out_specs=pl.BlockSpec((R, HDIM), lambda i: (i, 0)),
        out_shape=jax.ShapeDtypeStruct((N_NODES, HDIM), jnp.float32),
    )(x, mask2d, p0, p1, W_in, b2d, U)


def kernel(x, x_mask, h, edge_index, W_in, b_in, U):
    src = edge_index[0].astype(jnp.int32)
    dst = edge_index[1].astype(jnp.int32)
    pad = EDGES_PAD - N_EDGES
    src = jnp.concatenate([src, jnp.zeros((pad,), jnp.int32)])
    dst = jnp.concatenate([dst, jnp.full((pad,), N_NODES, jnp.int32)])
    src = src.reshape(NC, NS, NCHUNKS, CHUNK)
    dst = dst.reshape(NC, NS, NCHUNKS, CHUNK)
    zeros = jnp.zeros((ACC_ROWS, HDIM), jnp.float32)

    partials = _sc_segment_sum(h, src, dst, zeros)

    mask2d = x_mask.reshape(N_NODES, 1)
    b2d = b_in.reshape(1, HDIM)
    return _dense_stage(x, mask2d, partials[0, :N_NODES], partials[1, :N_NODES],
                        W_in, b2d, U)


# speedup vs baseline: 7.8465x; 2.1235x over previous
import functools

import jax
import jax.numpy as jnp
from jax import lax
from jax.experimental import pallas as pl
from jax.experimental.pallas import tpu as pltpu
from jax.experimental.pallas import tpu_sc as plsc

N_NODES = 10000
N_EDGES = 320000
HDIM = 128

NC = 2
NS = 16
CHUNK = 128
NCHUNKS = 79
EDGES_PAD = NC * NS * NCHUNKS * CHUNK
ACC_ROWS = 10112
ZROWS = ACC_ROWS // NS


def _sc_segment_sum(h, src, dst, zeros):
    mesh = plsc.VectorSubcoreMesh(core_axis_name="c", subcore_axis_name="s")

    @functools.partial(
        pl.kernel,
        out_type=jax.ShapeDtypeStruct((NC, ACC_ROWS, HDIM), jnp.float32),
        mesh=mesh,
        scratch_types=[
            pltpu.VMEM((NCHUNKS, CHUNK), jnp.int32),
            pltpu.VMEM((NCHUNKS, CHUNK), jnp.int32),
            pltpu.VMEM((CHUNK, HDIM), jnp.float32),
            pltpu.VMEM_SHARED((ACC_ROWS, HDIM), jnp.float32),
            pltpu.SemaphoreType.DMA,
        ],
    )
    def k(h_hbm, src_hbm, dst_hbm, zero_hbm, out_hbm, src_v, dst_v, rows_v, acc, sem):
        cid = lax.axis_index("c")
        sid = lax.axis_index("s")

        pltpu.sync_copy(zero_hbm.at[pl.ds(sid * ZROWS, ZROWS)],
                        acc.at[pl.ds(sid * ZROWS, ZROWS)])
        pltpu.sync_copy(src_hbm.at[cid, sid], src_v)
        pltpu.sync_copy(dst_hbm.at[cid, sid], dst_v)
        plsc.subcore_barrier()

        def body(j, carry):
            pltpu.async_copy(h_hbm.at[src_v.at[j]], rows_v, sem).wait()
            pltpu.sync_copy(rows_v, acc.at[dst_v.at[j]], add=True)
            return carry

        lax.fori_loop(0, NCHUNKS, body, 0, unroll=False)

        plsc.subcore_barrier()
        pltpu.sync_copy(acc.at[pl.ds(sid * ZROWS, ZROWS)],
                        out_hbm.at[cid, pl.ds(sid * ZROWS, ZROWS)])

    return k(h, src, dst, zeros)


def _dense_body(x_ref, m_ref, p0_ref, p1_ref, w_ref, b_ref, u_ref, o_ref):
    hsum = p0_ref[...] + p1_ref[...]
    h_aggr = jnp.dot(hsum, u_ref[...], preferred_element_type=jnp.float32)
    h_in = jnp.dot(x_ref[...], w_ref[...], preferred_element_type=jnp.float32)
    h_in = (h_in + b_ref[...]) * m_ref[...]
    o_ref[...] = jnp.tanh(h_in + h_aggr)


def _dense_stage(x, mask2d, p0, p1, W_in, b2d, U):
    R = 1000
    grid = (N_NODES // R,)
    return pl.pallas_call(
        _dense_body,
        grid=grid,
        in_specs=[
            pl.BlockSpec((R, HDIM), lambda i: (i, 0)),
            pl.BlockSpec((R, 1), lambda i: (i, 0)),
            pl.BlockSpec((R, HDIM), lambda i: (i, 0)),
            pl.BlockSpec((R, HDIM), lambda i: (i, 0)),
            pl.BlockSpec((HDIM, HDIM), lambda i: (0, 0)),
            pl.BlockSpec((1, HDIM), lambda i: (0, 0)),
            pl.BlockSpec((HDIM, HDIM), lambda i: (0, 0)),
        ],
        out_specs=pl.BlockSpec((R, HDIM), lambda i: (i, 0)),
        out_shape=jax.ShapeDtypeStruct((N_NODES, HDIM), jnp.float32),
    )(x, mask2d, p0, p1, W_in, b2d, U)


def kernel(x, x_mask, h, edge_index, W_in, b_in, U):
    src = edge_index[0].astype(jnp.int32)
    dst = edge_index[1].astype(jnp.int32)
    pad = EDGES_PAD - N_EDGES
    p = jnp.arange(pad)
    src = jnp.concatenate([src, (p % N_NODES).astype(jnp.int32)])
    dst = jnp.concatenate([dst, (N_NODES + (p // (NC * NS)) % (ACC_ROWS - N_NODES)
                                 ).astype(jnp.int32)])
    src = src.reshape(-1, NC * NS).T.reshape(NC, NS, NCHUNKS, CHUNK)
    dst = dst.reshape(-1, NC * NS).T.reshape(NC, NS, NCHUNKS, CHUNK)
    zeros = jnp.zeros((ACC_ROWS, HDIM), jnp.float32)

    partials = _sc_segment_sum(h, src, dst, zeros)

    mask2d = x_mask.reshape(N_NODES, 1)
    b2d = b_in.reshape(1, HDIM)
    return _dense_stage(x, mask2d, partials[0, :N_NODES], partials[1, :N_NODES],
                        W_in, b2d, U)


# speedup vs baseline: 11.4102x; 1.4542x over previous
import functools

import jax
import jax.numpy as jnp
from jax import lax
from jax.experimental import pallas as pl
from jax.experimental.pallas import tpu as pltpu
from jax.experimental.pallas import tpu_sc as plsc

N_NODES = 10000
N_EDGES = 320000
HDIM = 128

NC = 2
NS = 16
LANES = 16
CHUNK = 128
NBUF = 2
NCHUNKS = 80
EDGES_PAD = NC * NS * NCHUNKS * CHUNK
ACC_ROWS = 10112
ZROWS = ACC_ROWS // NS


def _sc_segment_sum(h, packed, zeros):
    mesh = plsc.VectorSubcoreMesh(core_axis_name="c", subcore_axis_name="s")

    @functools.partial(
        pl.kernel,
        out_type=jax.ShapeDtypeStruct((NC, ACC_ROWS, HDIM), jnp.float32),
        mesh=mesh,
        scratch_types=[
            pltpu.VMEM((NCHUNKS, CHUNK), jnp.int32),
            pltpu.VMEM((NBUF, CHUNK), jnp.int32),
            pltpu.VMEM((NBUF, CHUNK), jnp.int32),
            pltpu.VMEM((NBUF, CHUNK, HDIM), jnp.float32),
            pltpu.VMEM_SHARED((ACC_ROWS, HDIM), jnp.float32),
            pltpu.SemaphoreType.DMA((NBUF,)),
        ],
    )
    def k(h_hbm, pk_hbm, zero_hbm, out_hbm, pk_v, sidx, didx, rows_v, acc, gsem):
        cid = lax.axis_index("c")
        sid = lax.axis_index("s")

        pltpu.sync_copy(zero_hbm.at[pl.ds(sid * ZROWS, ZROWS)],
                        acc.at[pl.ds(sid * ZROWS, ZROWS)])
        pltpu.sync_copy(pk_hbm.at[cid, sid], pk_v)
        plsc.subcore_barrier()

        def unpack(j, b):
            for kk in range(CHUNK // LANES):
                pk = pk_v[j, pl.ds(kk * LANES, LANES)]
                sidx[b, pl.ds(kk * LANES, LANES)] = lax.bitwise_and(pk, 0xFFFF)
                didx[b, pl.ds(kk * LANES, LANES)] = lax.shift_right_logical(pk, 16)

        for b in range(NBUF):
            unpack(b, b)
            pltpu.async_copy(h_hbm.at[sidx.at[b]], rows_v.at[b], gsem.at[b])

        def body(g, carry):
            for b in range(NBUF):
                j = g * NBUF + b
                pltpu.make_async_copy(h_hbm.at[sidx.at[b]], rows_v.at[b],
                                      gsem.at[b]).wait()
                pltpu.sync_copy(rows_v.at[b], acc.at[didx.at[b]], add=True)

                @pl.when(g < NCHUNKS // NBUF - 1)
                def _():
                    unpack(j + NBUF, b)
                    pltpu.async_copy(h_hbm.at[sidx.at[b]], rows_v.at[b],
                                     gsem.at[b])
            return carry

        lax.fori_loop(0, NCHUNKS // NBUF, body, 0, unroll=False)

        plsc.subcore_barrier()
        pltpu.sync_copy(acc.at[pl.ds(sid * ZROWS, ZROWS)],
                        out_hbm.at[cid, pl.ds(sid * ZROWS, ZROWS)])

    return k(h, packed, zeros)


def _dense_body(x_ref, m_ref, p0_ref, p1_ref, w_ref, b_ref, u_ref, o_ref):
    hsum = p0_ref[...] + p1_ref[...]
    h_aggr = jnp.dot(hsum, u_ref[...], preferred_element_type=jnp.float32)
    h_in = jnp.dot(x_ref[...], w_ref[...], preferred_element_type=jnp.float32)
    h_in = (h_in + b_ref[...]) * m_ref[...]
    o_ref[...] = jnp.tanh(h_in + h_aggr)


def _dense_stage(x, mask2d, p0, p1, W_in, b2d, U):
    R = 1000
    grid = (N_NODES // R,)
    return pl.pallas_call(
        _dense_body,
        grid=grid,
        in_specs=[
            pl.BlockSpec((R, HDIM), lambda i: (i, 0)),
            pl.BlockSpec((R, 1), lambda i: (i, 0)),
            pl.BlockSpec((R, HDIM), lambda i: (i, 0)),
            pl.BlockSpec((R, HDIM), lambda i: (i, 0)),
            pl.BlockSpec((HDIM, HDIM), lambda i: (0, 0)),
            pl.BlockSpec((1, HDIM), lambda i: (0, 0)),
            pl.BlockSpec((HDIM, HDIM), lambda i: (0, 0)),
        ],
        out_specs=pl.BlockSpec((R, HDIM), lambda i: (i, 0)),
        out_shape=jax.ShapeDtypeStruct((N_NODES, HDIM), jnp.float32),
    )(x, mask2d, p0, p1, W_in, b2d, U)


def kernel(x, x_mask, h, edge_index, W_in, b_in, U):
    src = edge_index[0].astype(jnp.int32)
    dst = edge_index[1].astype(jnp.int32)
    pad = EDGES_PAD - N_EDGES
    p = jnp.arange(pad)
    src = jnp.concatenate([src, (p % N_NODES).astype(jnp.int32)])
    dst = jnp.concatenate([dst, (N_NODES + (p // (NC * NS)) % (ACC_ROWS - N_NODES)
                                 ).astype(jnp.int32)])
    packed = src | (dst << 16)
    packed = packed.reshape(-1, NC * NS).T.reshape(NC, NS, NCHUNKS, CHUNK)
    zeros = jnp.zeros((ACC_ROWS, HDIM), jnp.float32)

    partials = _sc_segment_sum(h, packed, zeros)

    mask2d = x_mask.reshape(N_NODES, 1)
    b2d = b_in.reshape(1, HDIM)
    return _dense_stage(x, mask2d, partials[0, :N_NODES], partials[1, :N_NODES],
                        W_in, b2d, U)


# speedup vs baseline: 12.0680x; 1.0576x over previous
import functools

import jax
import jax.numpy as jnp
from jax import lax
from jax.experimental import pallas as pl
from jax.experimental.pallas import tpu as pltpu
from jax.experimental.pallas import tpu_sc as plsc

N_NODES = 10000
N_EDGES = 320000
HDIM = 128

NC = 2
NS = 16
LANES = 16
CHUNK = 128
NBUF = 2
NCHUNKS = 80
EDGES_PAD = NC * NS * NCHUNKS * CHUNK
ACC_ROWS = 10112
ZROWS = ACC_ROWS // NS


def _sc_segment_sum(h, packed, zeros):
    mesh = plsc.VectorSubcoreMesh(core_axis_name="c", subcore_axis_name="s")

    @functools.partial(
        pl.kernel,
        out_type=jax.ShapeDtypeStruct((NC, ACC_ROWS, HDIM), jnp.float32),
        mesh=mesh,
        scratch_types=[
            pltpu.VMEM((NCHUNKS, CHUNK), jnp.int32),
            pltpu.VMEM((NBUF, CHUNK), jnp.int32),
            pltpu.VMEM((NBUF, CHUNK), jnp.int32),
            pltpu.VMEM((NBUF, CHUNK, HDIM), jnp.float32),
            pltpu.VMEM_SHARED((ACC_ROWS, HDIM), jnp.float32),
            pltpu.SemaphoreType.DMA((NBUF,)),
        ],
    )
    def k(h_hbm, pk_hbm, zero_hbm, out_hbm, pk_v, sidx, didx, rows_v, acc, gsem):
        cid = lax.axis_index("c")
        sid = lax.axis_index("s")

        pltpu.sync_copy(zero_hbm.at[pl.ds(sid * ZROWS, ZROWS)],
                        acc.at[pl.ds(sid * ZROWS, ZROWS)])
        pltpu.sync_copy(pk_hbm.at[cid, sid], pk_v)
        plsc.subcore_barrier()

        def unpack(j, b):
            for kk in range(CHUNK // LANES):
                pk = pk_v[j, pl.ds(kk * LANES, LANES)]
                sidx[b, pl.ds(kk * LANES, LANES)] = lax.bitwise_and(pk, 0xFFFF)
                didx[b, pl.ds(kk * LANES, LANES)] = lax.shift_right_logical(pk, 16)

        for b in range(NBUF):
            unpack(b, b)
            pltpu.async_copy(h_hbm.at[sidx.at[b]], rows_v.at[b], gsem.at[b])

        def body(g, carry):
            for b in range(NBUF):
                j = g * NBUF + b
                pltpu.make_async_copy(h_hbm.at[sidx.at[b]], rows_v.at[b],
                                      gsem.at[b]).wait()
                pltpu.sync_copy(rows_v.at[b], acc.at[didx.at[b]], add=True)

                @pl.when(g < NCHUNKS // NBUF - 1)
                def _():
                    unpack(j + NBUF, b)
                    pltpu.async_copy(h_hbm.at[sidx.at[b]], rows_v.at[b],
                                     gsem.at[b])
            return carry

        lax.fori_loop(0, NCHUNKS // NBUF, body, 0, unroll=False)

        plsc.subcore_barrier()
        pltpu.sync_copy(acc.at[pl.ds(sid * ZROWS, ZROWS)],
                        out_hbm.at[cid, pl.ds(sid * ZROWS, ZROWS)])

    return k(h, packed, zeros)


def _dense_body(x_ref, m_ref, p0_ref, p1_ref, w_ref, b_ref, u_ref, o_ref):
    hsum = p0_ref[...] + p1_ref[...]
    h_aggr = jnp.dot(hsum, u_ref[...], preferred_element_type=jnp.float32)
    h_in = jnp.dot(x_ref[...], w_ref[...], preferred_element_type=jnp.float32)
    h_in = (h_in + b_ref[...]) * m_ref[...]
    o_ref[...] = jnp.tanh(h_in + h_aggr)


def _dense_stage(x, mask2d, p0, p1, W_in, b2d, U):
    R = 1000
    grid = (N_NODES // R,)
    return pl.pallas_call(
        _dense_body,
        grid=grid,
        in_specs=[
            pl.BlockSpec((R, HDIM), lambda i: (i, 0)),
            pl.BlockSpec((R, 1), lambda i: (i, 0)),
            pl.BlockSpec((R, HDIM), lambda i: (i, 0)),
            pl.BlockSpec((R, HDIM), lambda i: (i, 0)),
            pl.BlockSpec((HDIM, HDIM), lambda i: (0, 0)),
            pl.BlockSpec((1, HDIM), lambda i: (0, 0)),
            pl.BlockSpec((HDIM, HDIM), lambda i: (0, 0)),
        ],
        out_specs=pl.BlockSpec((R, HDIM), lambda i: (i, 0)),
        out_shape=jax.ShapeDtypeStruct((N_NODES, HDIM), jnp.float32),
    )(x, mask2d, p0, p1, W_in, b2d, U)


def kernel(x, x_mask, h, edge_index, W_in, b_in, U):
    src = edge_index[0].astype(jnp.int32)
    dst = edge_index[1].astype(jnp.int32)
    pad = EDGES_PAD - N_EDGES
    p = jnp.arange(pad)
    src = jnp.concatenate([src, (p % N_NODES).astype(jnp.int32)])
    dst = jnp.concatenate([dst, (N_NODES + p % (ACC_ROWS - N_NODES)
                                 ).astype(jnp.int32)])
    packed = src | (dst << 16)
    packed = packed.reshape(NC, NS, NCHUNKS, CHUNK)
    zeros = jnp.zeros((ACC_ROWS, HDIM), jnp.float32)

    partials = _sc_segment_sum(h, packed, zeros)

    mask2d = x_mask.reshape(N_NODES, 1)
    b2d = b_in.reshape(1, HDIM)
    return _dense_stage(x, mask2d, partials[0, :N_NODES], partials[1, :N_NODES],
                        W_in, b2d, U)
